# baseline (speedup 1.0000x reference)
import functools

import jax
import jax.numpy as jnp
from jax import lax
from jax.experimental import pallas as pl
from jax.experimental.pallas import tpu as pltpu
from jax.experimental.pallas import tpu_sc as plsc

N = 10000
C = 128
NP = 10240
NPAD = NP - N
E = 320000
NW = 32
B = 128
NB = 80
EPW = NB * B
EP = NW * EPW
STRIPE = NP // 16
HNP = NP // 2
HSTRIPE = HNP // 16
SLOPE = (1.0 / 8.0 + 1.0 / 3.0) / 2.0


def _dense_body(x_ref, pw_ref, sc_ref, w0_ref, wu_ref, uu_ref, bu_ref,
                wr_ref, ur_ref, br_ref, wh_ref, uh_ref, bh_ref, hist_ref,
                xw_ref, xt_ref, xk_ref):
    xv = x_ref[...]
    pw = pw_ref[...]
    xt = lax.dot_general(xv, pw, (((1,), (1,)), ((), ())))
    xt_ref[...] = xt
    scv = sc_ref[...]
    s3 = lax.dot_general(xt.reshape(NP // 128, 128, C), scv,
                         (((2,), (0,)), ((), ())))
    smat = s3[:, :, 0]
    rio = lax.broadcasted_iota(jnp.int32, (NP // 128, 128), 0)
    cio = lax.broadcasted_iota(jnp.int32, (NP // 128, 128), 1)
    flat = rio * 128 + cio
    smat = jnp.where(flat >= N, -1e30, smat)
    lane = lax.broadcasted_iota(jnp.int32, (1, 128), 1)

    def tk_body(j, carry):
        s, vals = carry
        m = jnp.max(s)
        node = jnp.min(jnp.where(s == m, flat, jnp.int32(2 ** 30)))
        s = jnp.where(flat == node, -2e30, s)
        vals = jnp.where(lane == j, m, vals)
        xk_ref[pl.ds(j, 1), :] = xt_ref[pl.ds(node, 1), :]
        return s, vals

    _, vals = lax.fori_loop(0, C, tk_body,
                            (smat, jnp.zeros((1, 128), jnp.float32)))
    snorm = jnp.sqrt(jnp.sum(scv * scv))
    tvals = jnp.tanh(vals / snorm)
    xk = xk_ref[...]
    w0 = w0_ref[...]

    def gate(wref, uref, bref, rhs):
        zw = lax.dot_general(wref[...], xk, (((1,), (1,)), ((), ()))) * tvals
        return zw + jnp.dot(uref[...], rhs) + bref[...]

    def sigm(v):
        return 1.0 / (1.0 + jnp.exp(-v))

    upd = sigm(gate(wu_ref, uu_ref, bu_ref, w0))
    rst = sigm(gate(wr_ref, ur_ref, br_ref, w0))
    hcap = jnp.tanh(gate(wh_ref, uh_ref, bh_ref, rst * w0))
    wnew = (1.0 - upd) * w0 + upd * hcap
    xw = jnp.dot(xt, wnew)
    dinv = _dinv3(hist_ref)
    xw_ref[...] = (xw.reshape(NP // 128, 128, C) * dinv).reshape(NP, C)


def _dense_call(x_pad, pool_W, scorer, W0, Wu, Uu, bu, Wr, Ur, br,
                Wh, Uh, bh, hist):
    return pl.pallas_call(
        _dense_body,
        out_shape=jax.ShapeDtypeStruct((NP, C), jnp.float32),
        scratch_shapes=[pltpu.VMEM((NP, C), jnp.float32),
                        pltpu.VMEM((C, C), jnp.float32)],
    )(x_pad, pool_W, scorer, W0, Wu, Uu, bu, Wr, Ur, br, Wh, Uh, bh, hist)


def _dinv3(hist_ref):
    h = hist_ref[...]
    deg = h[0] + h[1] + 1.0
    return lax.rsqrt(deg)[:, :, None]


def _scale_body(xw_ref, hist_ref, y_ref):
    dinv = _dinv3(hist_ref)
    y3 = xw_ref[...].reshape(NP // 128, 128, C) * dinv
    y_ref[...] = y3.reshape(NP, C)


def _scale_call(xw, hist):
    return pl.pallas_call(
        _scale_body,
        out_shape=jax.ShapeDtypeStruct((NP, C), jnp.float32),
    )(xw, hist)


def _final_body(s_ref, y_ref, hist_ref, o_ref):
    dinv = _dinv3(hist_ref)
    sv = s_ref[...]
    acc = sv[0] + sv[1] + y_ref[...]
    o = (acc.reshape(NP // 128, 128, C) * dinv).reshape(NP, C)
    o_ref[...] = jnp.where(o >= 0.0, o, SLOPE * o)


def _final_call(spart, y, hist):
    return pl.pallas_call(
        _final_body,
        out_shape=jax.ShapeDtypeStruct((NP, C), jnp.float32),
    )(spart, y, hist)


@functools.cache
def _sc_hist_kernel():
    mesh = plsc.VectorSubcoreMesh(core_axis_name="c", subcore_axis_name="s")
    return pl.kernel(
        _sc_hist_body,
        out_type=jax.ShapeDtypeStruct((2, NP, C), jnp.float32),
        mesh=mesh,
        scratch_types=[
            pltpu.VMEM((B,), jnp.int32),
            pltpu.VMEM((B,), jnp.int32),
            pltpu.VMEM((B, C), jnp.float32),
            pltpu.VMEM_SHARED((NP, C), jnp.float32),
            pltpu.SemaphoreType.DMA,
            pltpu.SemaphoreType.DMA,
        ],
    )


def _sc_hist_body(dst_hbm, ones_hbm, zeros_hbm, out_hbm,
                  idxa, idxb, ones_v, hist_sh, sem, semb):
    c = lax.axis_index("c")
    s = lax.axis_index("s")
    wid = s * 2 + c
    pltpu.sync_copy(zeros_hbm, hist_sh.at[pl.ds(s * STRIPE, STRIPE)])
    pltpu.sync_copy(ones_hbm, ones_v)
    plsc.subcore_barrier()
    base = wid * EPW
    pltpu.sync_copy(dst_hbm.at[pl.ds(base, B)], idxa)
    pltpu.async_copy(ones_v, hist_sh.at[idxa], sem, add=True)

    def body(i, carry):
        j = i * 2
        pltpu.sync_copy(dst_hbm.at[pl.ds(base + (j + 1) * B, B)], idxb)
        pltpu.async_copy(ones_v, hist_sh.at[idxb], semb, add=True)
        pltpu.make_async_copy(ones_v, hist_sh.at[idxa], sem).wait()

        @pl.when(j + 2 < NB)
        def _():
            pltpu.sync_copy(dst_hbm.at[pl.ds(base + (j + 2) * B, B)], idxa)
            pltpu.async_copy(ones_v, hist_sh.at[idxa], sem, add=True)

        pltpu.make_async_copy(ones_v, hist_sh.at[idxb], semb).wait()
        return carry

    lax.fori_loop(0, NB // 2, body, 0)
    plsc.subcore_barrier()
    pltpu.sync_copy(hist_sh.at[pl.ds(s * STRIPE, STRIPE)],
                    out_hbm.at[c, pl.ds(s * STRIPE, STRIPE)])


@functools.cache
def _sc_scatter_kernel():
    mesh = plsc.VectorSubcoreMesh(core_axis_name="c", subcore_axis_name="s")
    return pl.kernel(
        _sc_scatter_body,
        out_type=jax.ShapeDtypeStruct((2, NP, C), jnp.float32),
        mesh=mesh,
        scratch_types=[
            pltpu.VMEM((2, B), jnp.int32),
            pltpu.VMEM((2, B), jnp.int32),
            pltpu.VMEM((B, C), jnp.float32),
            pltpu.VMEM((B, C), jnp.float32),
            pltpu.VMEM_SHARED((NP, C), jnp.float32),
            pltpu.SemaphoreType.DMA,
            pltpu.SemaphoreType.DMA,
        ],
    )


def _sc_scatter_body(y_hbm, sd_hbm, zrow_hbm, out_hbm,
                     idxa, idxb, rows0, rows1, acc_sh, sem0, sem1):
    c = lax.axis_index("c")
    s = lax.axis_index("s")
    wid = s * 2 + c
    pltpu.sync_copy(zrow_hbm, acc_sh.at[pl.ds(s * STRIPE, STRIPE)])
    plsc.subcore_barrier()
    pltpu.sync_copy(sd_hbm.at[wid, 0], idxa)
    pltpu.async_copy(y_hbm.at[idxa.at[0]], rows0, sem0)

    def body(i, carry):
        j = i * 2
        pltpu.sync_copy(sd_hbm.at[wid, j + 1], idxb)
        pltpu.async_copy(y_hbm.at[idxb.at[0]], rows1, sem1)
        pltpu.make_async_copy(y_hbm.at[idxa.at[0]], rows0, sem0).wait()
        pltpu.sync_copy(rows0, acc_sh.at[idxa.at[1]], add=True)

        @pl.when(j + 2 < NB)
        def _():
            pltpu.sync_copy(sd_hbm.at[wid, j + 2], idxa)
            pltpu.async_copy(y_hbm.at[idxa.at[0]], rows0, sem0)

        pltpu.make_async_copy(y_hbm.at[idxb.at[0]], rows1, sem1).wait()
        pltpu.sync_copy(rows1, acc_sh.at[idxb.at[1]], add=True)
        return carry

    lax.fori_loop(0, NB // 2, body, 0)
    plsc.subcore_barrier()
    pltpu.sync_copy(acc_sh.at[pl.ds(s * STRIPE, STRIPE)],
                    out_hbm.at[c, pl.ds(s * STRIPE, STRIPE)])


def kernel(x, edge_index, pool_W, scorer, W0, Wu, Uu, bu, Wr, Ur, br,
           Wh, Uh, bh):
    x_pad = jnp.pad(x, ((0, NPAD), (0, 0)))
    padi = N + (jnp.arange(EP - E, dtype=jnp.int32) % NPAD)
    src = jnp.concatenate([edge_index[0], padi])
    dst = jnp.concatenate([edge_index[1], padi])
    ones = jnp.ones((B, C), jnp.float32)
    zr = jnp.zeros((STRIPE, C), jnp.float32)

    hist_full = _sc_hist_kernel()(dst, ones, zr)
    hist = hist_full[:, :, 0].reshape(2, NP // 128, 128)
    y = _dense_call(x_pad, pool_W, scorer, W0, Wu, Uu, bu,
                    Wr, Ur, br, Wh, Uh, bh, hist)
    sd = jnp.stack([src.reshape(NW, NB, B), dst.reshape(NW, NB, B)], axis=2)
    spart = _sc_scatter_kernel()(y, sd, zr)
    out = _final_call(spart, y, hist)
    return out[:N]


# speedup vs baseline: 1.1697x; 1.1697x over previous
import functools

import jax
import jax.numpy as jnp
from jax import lax
from jax.experimental import pallas as pl
from jax.experimental.pallas import tpu as pltpu
from jax.experimental.pallas import tpu_sc as plsc

N = 10000
C = 128
NP = 10240
NPAD = NP - N
E = 320000
NW = 32
B = 128
NB = 80
EPW = NB * B
EP = NW * EPW
STRIPE = NP // 16
HNP = NP // 2
HSTRIPE = HNP // 16
SLOPE = (1.0 / 8.0 + 1.0 / 3.0) / 2.0


def _dense_body(x_ref, pw_ref, sc_ref, w0_ref, wu_ref, uu_ref, bu_ref,
                wr_ref, ur_ref, br_ref, wh_ref, uh_ref, bh_ref,
                xw_ref, xt_ref, xk_ref):
    xv = x_ref[...]
    pw = pw_ref[...]
    xt = lax.dot_general(xv, pw, (((1,), (1,)), ((), ())))
    xt_ref[...] = xt
    scv = sc_ref[...]
    s3 = lax.dot_general(xt.reshape(NP // 128, 128, C), scv,
                         (((2,), (0,)), ((), ())))
    smat = s3[:, :, 0]
    rio = lax.broadcasted_iota(jnp.int32, (NP // 128, 128), 0)
    cio = lax.broadcasted_iota(jnp.int32, (NP // 128, 128), 1)
    flat = rio * 128 + cio
    smat = jnp.where(flat >= N, -1e30, smat)
    lane = lax.broadcasted_iota(jnp.int32, (1, 128), 1)

    def tk_body(j, carry):
        s, vals = carry
        m = jnp.max(s)
        node = jnp.min(jnp.where(s == m, flat, jnp.int32(2 ** 30)))
        s = jnp.where(flat == node, -2e30, s)
        vals = jnp.where(lane == j, m, vals)
        xk_ref[pl.ds(j, 1), :] = xt_ref[pl.ds(node, 1), :]
        return s, vals

    _, vals = lax.fori_loop(0, C, tk_body,
                            (smat, jnp.zeros((1, 128), jnp.float32)))
    snorm = jnp.sqrt(jnp.sum(scv * scv))
    tvals = jnp.tanh(vals / snorm)
    xk = xk_ref[...]
    w0 = w0_ref[...]

    def gate(wref, uref, bref, rhs):
        zw = lax.dot_general(wref[...], xk, (((1,), (1,)), ((), ()))) * tvals
        return zw + jnp.dot(uref[...], rhs) + bref[...]

    def sigm(v):
        return 1.0 / (1.0 + jnp.exp(-v))

    upd = sigm(gate(wu_ref, uu_ref, bu_ref, w0))
    rst = sigm(gate(wr_ref, ur_ref, br_ref, w0))
    hcap = jnp.tanh(gate(wh_ref, uh_ref, bh_ref, rst * w0))
    wnew = (1.0 - upd) * w0 + upd * hcap
    xw_ref[...] = jnp.dot(xt, wnew)


def _dense_call(x_pad, pool_W, scorer, W0, Wu, Uu, bu, Wr, Ur, br, Wh, Uh, bh):
    return pl.pallas_call(
        _dense_body,
        out_shape=jax.ShapeDtypeStruct((NP, C), jnp.float32),
        scratch_shapes=[pltpu.VMEM((NP, C), jnp.float32),
                        pltpu.VMEM((C, C), jnp.float32)],
    )(x_pad, pool_W, scorer, W0, Wu, Uu, bu, Wr, Ur, br, Wh, Uh, bh)


def _dinv3(hist_ref):
    h = hist_ref[...]
    deg = h[0] + h[1] + 1.0
    return lax.rsqrt(deg)[:, :, None]


def _scale_body(xw_ref, hist_ref, y_ref):
    dinv = _dinv3(hist_ref)
    y3 = xw_ref[...].reshape(NP // 128, 128, C) * dinv
    y_ref[...] = y3.reshape(NP, C)


def _scale_call(xw, hist):
    return pl.pallas_call(
        _scale_body,
        out_shape=jax.ShapeDtypeStruct((NP, C), jnp.float32),
    )(xw, hist)


def _final_body(s_ref, y_ref, hist_ref, o_ref):
    dinv = _dinv3(hist_ref)
    sv = s_ref[...]
    acc = sv[0] + sv[1] + y_ref[...]
    o = (acc.reshape(NP // 128, 128, C) * dinv).reshape(NP, C)
    o_ref[...] = jnp.where(o >= 0.0, o, SLOPE * o)


def _final_call(spart, y, hist):
    return pl.pallas_call(
        _final_body,
        out_shape=jax.ShapeDtypeStruct((NP, C), jnp.float32),
    )(spart, y, hist)


@functools.cache
def _sc_hist_kernel():
    mesh = plsc.VectorSubcoreMesh(core_axis_name="c", subcore_axis_name="s")
    return pl.kernel(
        _sc_hist_body,
        out_type=jax.ShapeDtypeStruct((2, NP, C), jnp.float32),
        mesh=mesh,
        scratch_types=[
            pltpu.VMEM((B,), jnp.int32),
            pltpu.VMEM((B,), jnp.int32),
            pltpu.VMEM((B, C), jnp.float32),
            pltpu.VMEM_SHARED((NP, C), jnp.float32),
            pltpu.SemaphoreType.DMA,
            pltpu.SemaphoreType.DMA,
        ],
    )


def _sc_hist_body(dst_hbm, ones_hbm, zeros_hbm, out_hbm,
                  idxa, idxb, ones_v, hist_sh, sem, semb):
    c = lax.axis_index("c")
    s = lax.axis_index("s")
    wid = s * 2 + c
    pltpu.sync_copy(zeros_hbm, hist_sh.at[pl.ds(s * STRIPE, STRIPE)])
    pltpu.sync_copy(ones_hbm, ones_v)
    plsc.subcore_barrier()
    base = wid * EPW
    pltpu.sync_copy(dst_hbm.at[pl.ds(base, B)], idxa)
    pltpu.async_copy(ones_v, hist_sh.at[idxa], sem, add=True)

    def body(i, carry):
        j = i * 2
        pltpu.sync_copy(dst_hbm.at[pl.ds(base + (j + 1) * B, B)], idxb)
        pltpu.async_copy(ones_v, hist_sh.at[idxb], semb, add=True)
        pltpu.make_async_copy(ones_v, hist_sh.at[idxa], sem).wait()

        @pl.when(j + 2 < NB)
        def _():
            pltpu.sync_copy(dst_hbm.at[pl.ds(base + (j + 2) * B, B)], idxa)
            pltpu.async_copy(ones_v, hist_sh.at[idxa], sem, add=True)

        pltpu.make_async_copy(ones_v, hist_sh.at[idxb], semb).wait()
        return carry

    lax.fori_loop(0, NB // 2, body, 0)
    plsc.subcore_barrier()
    pltpu.sync_copy(hist_sh.at[pl.ds(s * STRIPE, STRIPE)],
                    out_hbm.at[c, pl.ds(s * STRIPE, STRIPE)])


@functools.cache
def _sc_scatter_kernel():
    mesh = plsc.VectorSubcoreMesh(core_axis_name="c", subcore_axis_name="s")
    return pl.kernel(
        _sc_scatter_body,
        out_type=jax.ShapeDtypeStruct((2, NP, C), jnp.float32),
        mesh=mesh,
        scratch_types=[
            pltpu.VMEM((2, B), jnp.int32),
            pltpu.VMEM((2, B), jnp.int32),
            pltpu.VMEM((B, C), jnp.float32),
            pltpu.VMEM((B, C), jnp.float32),
            pltpu.VMEM_SHARED((NP, C), jnp.float32),
            pltpu.SemaphoreType.DMA,
            pltpu.SemaphoreType.DMA,
        ],
    )


def _sc_scatter_body(y_hbm, sd_hbm, zrow_hbm, out_hbm,
                     idxa, idxb, rows0, rows1, acc_sh, sem0, sem1):
    c = lax.axis_index("c")
    s = lax.axis_index("s")
    wid = s * 2 + c
    pltpu.sync_copy(zrow_hbm, acc_sh.at[pl.ds(s * STRIPE, STRIPE)])
    plsc.subcore_barrier()
    pltpu.sync_copy(sd_hbm.at[wid, 0], idxa)
    pltpu.async_copy(y_hbm.at[idxa.at[0]], rows0, sem0)

    def body(i, carry):
        j = i * 2
        pltpu.sync_copy(sd_hbm.at[wid, j + 1], idxb)
        pltpu.async_copy(y_hbm.at[idxb.at[0]], rows1, sem1)
        pltpu.make_async_copy(y_hbm.at[idxa.at[0]], rows0, sem0).wait()
        pltpu.sync_copy(rows0, acc_sh.at[idxa.at[1]], add=True)

        @pl.when(j + 2 < NB)
        def _():
            pltpu.sync_copy(sd_hbm.at[wid, j + 2], idxa)
            pltpu.async_copy(y_hbm.at[idxa.at[0]], rows0, sem0)

        pltpu.make_async_copy(y_hbm.at[idxb.at[0]], rows1, sem1).wait()
        pltpu.sync_copy(rows1, acc_sh.at[idxb.at[1]], add=True)
        return carry

    lax.fori_loop(0, NB // 2, body, 0)
    plsc.subcore_barrier()
    pltpu.sync_copy(acc_sh.at[pl.ds(s * STRIPE, STRIPE)],
                    out_hbm.at[c, pl.ds(s * STRIPE, STRIPE)])


def kernel(x, edge_index, pool_W, scorer, W0, Wu, Uu, bu, Wr, Ur, br,
           Wh, Uh, bh):
    x_pad = jnp.pad(x, ((0, NPAD), (0, 0)))
    padi = N + (jnp.arange(EP - E, dtype=jnp.int32) % NPAD)
    src = jnp.concatenate([edge_index[0], padi])
    dst = jnp.concatenate([edge_index[1], padi])
    ones = jnp.ones((B, C), jnp.float32)
    zr = jnp.zeros((STRIPE, C), jnp.float32)

    hist_full = _sc_hist_kernel()(dst, ones, zr)
    xw = _dense_call(x_pad, pool_W, scorer, W0, Wu, Uu, bu,
                     Wr, Ur, br, Wh, Uh, bh)
    hist = hist_full[:, :, 0].reshape(2, NP // 128, 128)
    y = _scale_call(xw, hist)
    sd = jnp.stack([src.reshape(NW, NB, B), dst.reshape(NW, NB, B)], axis=2)
    spart = _sc_scatter_kernel()(y, sd, zr)
    out = _final_call(spart, y, hist)
    return out[:N]
